# baseline (speedup 1.0000x reference)
import functools

import jax
import jax.numpy as jnp
from jax import lax
from jax.experimental import pallas as pl
from jax.experimental.pallas import tpu as pltpu
from jax.experimental.pallas import tpu_sc as plsc

VOCAB = 32768
HIDDEN = 256
B, L = 64, 32
NTOK = B * L

VB = 2048


def _gather_sc(embed_table, flat_ids):
    n = flat_ids.shape[0]
    info = plsc.get_sparse_core_info()
    nw = info.num_cores * info.num_subcores
    b_per_w = n // nw
    mesh = plsc.VectorSubcoreMesh(core_axis_name="c", subcore_axis_name="s")

    @functools.partial(
        pl.kernel,
        out_type=jax.ShapeDtypeStruct((n, HIDDEN), jnp.float32),
        mesh=mesh,
        scratch_types=[
            pltpu.VMEM((b_per_w,), jnp.int32),
            pltpu.VMEM((b_per_w, HIDDEN), jnp.float32),
            pltpu.SemaphoreType.DMA,
        ],
    )
    def gather_kernel(table_hbm, idx_hbm, out_hbm, idx_v, rows_v, sem):
        wid = lax.axis_index("s") * info.num_cores + lax.axis_index("c")
        base = wid * b_per_w
        pltpu.sync_copy(idx_hbm.at[pl.ds(base, b_per_w)], idx_v)
        pltpu.async_copy(table_hbm.at[idx_v], rows_v, sem).wait()
        pltpu.sync_copy(rows_v, out_hbm.at[pl.ds(base, b_per_w)])

    return gather_kernel(embed_table, flat_ids)


HALF = NTOK // 2


def _mm_kernel(h_ref, w_ref, out_ref):
    hb = h_ref[...].astype(jnp.bfloat16)
    wb = w_ref[...].astype(jnp.bfloat16)
    out_ref[...] = lax.dot_general(
        hb, wb, (((1,), (1,)), ((), ())), preferred_element_type=jnp.float32
    )


def _mm_half0(h, head_w):
    return pl.pallas_call(
        _mm_kernel,
        grid=(VOCAB // VB,),
        in_specs=[
            pl.BlockSpec((HALF, HIDDEN), lambda i: (0, 0)),
            pl.BlockSpec((VB, HIDDEN), lambda i: (i, 0)),
        ],
        out_specs=pl.BlockSpec((HALF, VB), lambda i: (0, i)),
        out_shape=jax.ShapeDtypeStruct((NTOK, VOCAB), jnp.float32),
    )(h, head_w)


def _mm_half1_kernel(h_ref, w_ref, carry_ref, out_ref):
    del carry_ref
    _mm_kernel(h_ref, w_ref, out_ref)


def _mm_half1(h, head_w, carry):
    return pl.pallas_call(
        _mm_half1_kernel,
        grid=(VOCAB // VB,),
        in_specs=[
            pl.BlockSpec((HALF, HIDDEN), lambda i: (0, 0)),
            pl.BlockSpec((VB, HIDDEN), lambda i: (i, 0)),
            pl.BlockSpec(memory_space=pl.ANY),
        ],
        out_specs=pl.BlockSpec((HALF, VB), lambda i: (1, i)),
        out_shape=jax.ShapeDtypeStruct((NTOK, VOCAB), jnp.float32),
        input_output_aliases={2: 0},
    )(h, head_w, carry)


def kernel(input_ids, embed_table, head_w):
    flat_ids = input_ids.reshape(NTOK).astype(jnp.int32)
    h0 = _gather_sc(embed_table, flat_ids[:HALF])
    h1 = _gather_sc(embed_table, flat_ids[HALF:])
    l0 = _mm_half0(h0, head_w)
    logits = _mm_half1(h1, head_w, l0)
    return logits.reshape(B, L, VOCAB)


# speedup vs baseline: 1.1034x; 1.1034x over previous
import functools

import jax
import jax.numpy as jnp
from jax import lax
from jax.experimental import pallas as pl
from jax.experimental.pallas import tpu as pltpu
from jax.experimental.pallas import tpu_sc as plsc

VOCAB = 32768
HIDDEN = 256
B, L = 64, 32
NTOK = B * L

VB = 2048


def _gather_sc(embed_table, flat_ids):
    n = flat_ids.shape[0]
    info = plsc.get_sparse_core_info()
    nw = info.num_cores * info.num_subcores
    b_per_w = n // nw
    mesh = plsc.VectorSubcoreMesh(core_axis_name="c", subcore_axis_name="s")

    @functools.partial(
        pl.kernel,
        out_type=jax.ShapeDtypeStruct((n, HIDDEN), jnp.float32),
        mesh=mesh,
        scratch_types=[
            pltpu.VMEM((b_per_w,), jnp.int32),
            pltpu.VMEM((b_per_w, HIDDEN), jnp.float32),
            pltpu.SemaphoreType.DMA,
        ],
    )
    def gather_kernel(table_hbm, idx_hbm, out_hbm, idx_v, rows_v, sem):
        wid = lax.axis_index("s") * info.num_cores + lax.axis_index("c")
        base = wid * b_per_w
        pltpu.sync_copy(idx_hbm.at[pl.ds(base, b_per_w)], idx_v)
        pltpu.async_copy(table_hbm.at[idx_v], rows_v, sem).wait()
        pltpu.sync_copy(rows_v, out_hbm.at[pl.ds(base, b_per_w)])

    return gather_kernel(embed_table, flat_ids)


TB = 512
VBB = 8192


def _mm_kernel(h_ref, w_ref, out_ref):
    hb = h_ref[...].astype(jnp.bfloat16)
    wb = w_ref[...].astype(jnp.bfloat16)
    out_ref[...] = lax.dot_general(
        hb, wb, (((1,), (1,)), ((), ())), preferred_element_type=jnp.float32
    )


def _matmul_tc(h, head_w):
    return pl.pallas_call(
        _mm_kernel,
        grid=(VOCAB // VBB, NTOK // TB),
        in_specs=[
            pl.BlockSpec((TB, HIDDEN), lambda v, t: (t, 0)),
            pl.BlockSpec((VBB, HIDDEN), lambda v, t: (v, 0)),
        ],
        out_specs=pl.BlockSpec((TB, VBB), lambda v, t: (t, v)),
        out_shape=jax.ShapeDtypeStruct((NTOK, VOCAB), jnp.float32),
    )(h, head_w)


def kernel(input_ids, embed_table, head_w):
    flat_ids = input_ids.reshape(NTOK).astype(jnp.int32)
    h = _gather_sc(embed_table, flat_ids)
    logits = _matmul_tc(h, head_w)
    return logits.reshape(B, L, VOCAB)


# speedup vs baseline: 1.1232x; 1.0179x over previous
import functools

import jax
import jax.numpy as jnp
from jax import lax
from jax.experimental import pallas as pl
from jax.experimental.pallas import tpu as pltpu
from jax.experimental.pallas import tpu_sc as plsc

VOCAB = 32768
HIDDEN = 256
B, L = 64, 32
NTOK = B * L

VB = 2048


def _gather_sc(embed_table, flat_ids):
    n = flat_ids.shape[0]
    info = plsc.get_sparse_core_info()
    nw = info.num_cores * info.num_subcores
    b_per_w = n // nw
    mesh = plsc.VectorSubcoreMesh(core_axis_name="c", subcore_axis_name="s")

    half = b_per_w // 2

    @functools.partial(
        pl.kernel,
        out_type=jax.ShapeDtypeStruct((n, HIDDEN), jnp.float32),
        mesh=mesh,
        scratch_types=[
            pltpu.VMEM((half,), jnp.int32),
            pltpu.VMEM((half,), jnp.int32),
            pltpu.VMEM((half, HIDDEN), jnp.float32),
            pltpu.VMEM((half, HIDDEN), jnp.float32),
            pltpu.SemaphoreType.DMA,
            pltpu.SemaphoreType.DMA,
            pltpu.SemaphoreType.DMA,
            pltpu.SemaphoreType.DMA,
        ],
    )
    def gather_kernel(table_hbm, idx_hbm, out_hbm, idx0, idx1, rows0, rows1,
                      s0, s1, s2, s3):
        wid = lax.axis_index("s") * info.num_cores + lax.axis_index("c")
        base = wid * b_per_w
        pltpu.sync_copy(idx_hbm.at[pl.ds(base, half)], idx0)
        g0 = pltpu.async_copy(table_hbm.at[idx0], rows0, s0)
        pltpu.sync_copy(idx_hbm.at[pl.ds(base + half, half)], idx1)
        g1 = pltpu.async_copy(table_hbm.at[idx1], rows1, s1)
        g0.wait()
        w0 = pltpu.async_copy(rows0, out_hbm.at[pl.ds(base, half)], s2)
        g1.wait()
        w1 = pltpu.async_copy(rows1, out_hbm.at[pl.ds(base + half, half)], s3)
        w0.wait()
        w1.wait()

    return gather_kernel(embed_table, flat_ids)


def _mm_kernel(h_ref, w_ref, out_ref):
    hb = h_ref[...].astype(jnp.bfloat16)
    wb = w_ref[...].astype(jnp.bfloat16)
    out_ref[...] = lax.dot_general(
        hb, wb, (((1,), (1,)), ((), ())), preferred_element_type=jnp.float32
    )


def _matmul_tc(h, head_w):
    return pl.pallas_call(
        _mm_kernel,
        grid=(VOCAB // VB,),
        in_specs=[
            pl.BlockSpec((NTOK, HIDDEN), lambda i: (0, 0)),
            pl.BlockSpec((VB, HIDDEN), lambda i: (i, 0)),
        ],
        out_specs=pl.BlockSpec((NTOK, VB), lambda i: (0, i)),
        out_shape=jax.ShapeDtypeStruct((NTOK, VOCAB), jnp.float32),
    )(h, head_w)


def kernel(input_ids, embed_table, head_w):
    flat_ids = input_ids.reshape(NTOK).astype(jnp.int32)
    h = _gather_sc(embed_table, flat_ids)
    logits = _matmul_tc(h, head_w)
    return logits.reshape(B, L, VOCAB)


# speedup vs baseline: 1.1317x; 1.0076x over previous
import functools

import jax
import jax.numpy as jnp
from jax import lax
from jax.experimental import pallas as pl
from jax.experimental.pallas import tpu as pltpu
from jax.experimental.pallas import tpu_sc as plsc

VOCAB = 32768
HIDDEN = 256
B, L = 64, 32
NTOK = B * L

VB = 2048


def _gather_sc(embed_table, flat_ids):
    n = flat_ids.shape[0]
    info = plsc.get_sparse_core_info()
    ncores = 1
    nw = ncores * info.num_subcores
    b_per_w = n // nw
    mesh = plsc.VectorSubcoreMesh(
        core_axis_name="c", subcore_axis_name="s", num_cores=ncores
    )

    half = b_per_w // 2

    @functools.partial(
        pl.kernel,
        out_type=jax.ShapeDtypeStruct((n, HIDDEN), jnp.float32),
        mesh=mesh,
        scratch_types=[
            pltpu.VMEM((half,), jnp.int32),
            pltpu.VMEM((half,), jnp.int32),
            pltpu.VMEM((half, HIDDEN), jnp.float32),
            pltpu.VMEM((half, HIDDEN), jnp.float32),
            pltpu.SemaphoreType.DMA,
            pltpu.SemaphoreType.DMA,
            pltpu.SemaphoreType.DMA,
            pltpu.SemaphoreType.DMA,
        ],
    )
    def gather_kernel(table_hbm, idx_hbm, out_hbm, idx0, idx1, rows0, rows1,
                      s0, s1, s2, s3):
        wid = lax.axis_index("s") * ncores + lax.axis_index("c")
        base = wid * b_per_w
        pltpu.sync_copy(idx_hbm.at[pl.ds(base, half)], idx0)
        g0 = pltpu.async_copy(table_hbm.at[idx0], rows0, s0)
        pltpu.sync_copy(idx_hbm.at[pl.ds(base + half, half)], idx1)
        g1 = pltpu.async_copy(table_hbm.at[idx1], rows1, s1)
        g0.wait()
        w0 = pltpu.async_copy(rows0, out_hbm.at[pl.ds(base, half)], s2)
        g1.wait()
        w1 = pltpu.async_copy(rows1, out_hbm.at[pl.ds(base + half, half)], s3)
        w0.wait()
        w1.wait()

    return gather_kernel(embed_table, flat_ids)


def _mm_kernel(h_ref, w_ref, out_ref):
    hb = h_ref[...].astype(jnp.bfloat16)
    wb = w_ref[...].astype(jnp.bfloat16)
    out_ref[...] = lax.dot_general(
        hb, wb, (((1,), (1,)), ((), ())), preferred_element_type=jnp.float32
    )


def _matmul_tc(h, head_w):
    return pl.pallas_call(
        _mm_kernel,
        grid=(VOCAB // VB,),
        in_specs=[
            pl.BlockSpec((NTOK, HIDDEN), lambda i: (0, 0)),
            pl.BlockSpec((VB, HIDDEN), lambda i: (i, 0)),
        ],
        out_specs=pl.BlockSpec((NTOK, VB), lambda i: (0, i)),
        out_shape=jax.ShapeDtypeStruct((NTOK, VOCAB), jnp.float32),
    )(h, head_w)


def kernel(input_ids, embed_table, head_w):
    flat_ids = input_ids.reshape(NTOK).astype(jnp.int32)
    h = _gather_sc(embed_table, flat_ids)
    logits = _matmul_tc(h, head_w)
    return logits.reshape(B, L, VOCAB)
